# baseline (speedup 1.0000x reference)
import functools

import jax
import jax.numpy as jnp
from jax import lax
from jax.experimental import pallas as pl
from jax.experimental.pallas import tpu as pltpu
from jax.experimental.pallas import tpu_sc as plsc

EMBED_D = 32
LANES = 16
EPS = 1e-5
CHUNK = 512
IDX_SUB = 128
K_SUB = CHUNK // IDX_SUB
UNROLL = 8
NBUF = 4


def _rsqrt(x):
    i = lax.bitcast_convert_type(x, jnp.int32)
    i = jnp.int32(0x5F3759DF) - (i >> 1)
    y = lax.bitcast_convert_type(i, jnp.float32)
    half = 0.5 * x
    for _ in range(3):
        y = y * (1.5 - half * y * y)
    return y


def _make_sc_kernel(n_rows, n_workers):
    rows_per_w = n_rows // n_workers
    n_chunks = rows_per_w // CHUNK
    assert n_chunks % NBUF == 0

    mesh = plsc.VectorSubcoreMesh(core_axis_name="c", subcore_axis_name="s")

    @functools.partial(
        pl.kernel,
        mesh=mesh,
        out_type=jax.ShapeDtypeStruct((n_rows, EMBED_D), jnp.float32),
        compiler_params=pltpu.CompilerParams(
            use_tc_tiling_on_sc=False, needs_layout_passes=False),
        scratch_types=(
            [pltpu.VMEM((K_SUB, IDX_SUB), jnp.int32) for _ in range(NBUF)]
            + [pltpu.VMEM((CHUNK, EMBED_D), jnp.float32) for _ in range(NBUF)]
            + [pltpu.VMEM((EMBED_D,), jnp.float32)] * 2
            + [pltpu.SemaphoreType.DMA] * (3 * NBUF)
        ),
    )
    def sc_kernel(x2d_hbm, table_hbm, gamma_hbm, beta_hbm, out_hbm, *scratch):
        idxb = scratch[:NBUF]
        rowsb = scratch[NBUF:2 * NBUF]
        gamma_v, beta_v = scratch[2 * NBUF:2 * NBUF + 2]
        isem = scratch[2 * NBUF + 2:2 * NBUF + 2 + NBUF]
        gsem = scratch[2 * NBUF + 2 + NBUF:2 * NBUF + 2 + 2 * NBUF]
        osem = scratch[2 * NBUF + 2 + 2 * NBUF:]

        wid = lax.axis_index("s") * 2 + lax.axis_index("c")
        wbase = wid * rows_per_w

        def cbase(c):
            return pl.multiple_of(wbase + c * CHUNK, CHUNK)

        def irow0(c):
            return pl.multiple_of(cbase(c) // IDX_SUB, K_SUB)

        def fire_idx(c, b):
            pltpu.async_copy(
                x2d_hbm.at[pl.ds(irow0(c), K_SUB)], idxb[b], isem[b])

        def wait_idx(c, b):
            pltpu.make_async_copy(
                x2d_hbm.at[pl.ds(irow0(c), K_SUB)], idxb[b], isem[b]).wait()

        def fire_gathers(b):
            for j in range(K_SUB):
                pltpu.async_copy(
                    table_hbm.at[idxb[b].at[j]],
                    rowsb[b].at[pl.ds(j * IDX_SUB, IDX_SUB)], gsem[b])

        def wait_gathers(b):
            for j in range(K_SUB):
                pltpu.make_async_copy(
                    table_hbm.at[idxb[b].at[j]],
                    rowsb[b].at[pl.ds(j * IDX_SUB, IDX_SUB)], gsem[b]).wait()

        def fire_out(c, b):
            pltpu.async_copy(
                rowsb[b], out_hbm.at[pl.ds(cbase(c), CHUNK)], osem[b])

        def wait_out(c, b):
            pltpu.make_async_copy(
                rowsb[b], out_hbm.at[pl.ds(cbase(c), CHUNK)], osem[b]).wait()

        pltpu.sync_copy(gamma_hbm, gamma_v)
        pltpu.sync_copy(beta_hbm, beta_v)
        g0 = gamma_v[pl.ds(0, LANES)]
        g1 = gamma_v[pl.ds(LANES, LANES)]
        b0 = beta_v[pl.ds(0, LANES)]
        b1 = beta_v[pl.ds(LANES, LANES)]

        iota = lax.iota(jnp.int32, LANES)
        shuf = [iota ^ sh for sh in (8, 4, 2, 1)]

        def allsum(v):
            for s in shuf:
                v = v + v.at[s].get(mode="promise_in_bounds")
            return v

        def compute(rows_v):
            def one_row(r):
                v0 = rows_v[r, pl.ds(0, LANES)]
                v1 = rows_v[r, pl.ds(LANES, LANES)]
                mean = allsum(v0 + v1) * (1.0 / EMBED_D)
                d0 = v0 - mean
                d1 = v1 - mean
                var = allsum(d0 * d0 + d1 * d1) * (1.0 / EMBED_D)
                rinv = _rsqrt(var + EPS)
                rows_v[r, pl.ds(0, LANES)] = d0 * rinv * g0 + b0
                rows_v[r, pl.ds(LANES, LANES)] = d1 * rinv * g1 + b1

            def row_body(r8, carry):
                for u in range(UNROLL):
                    one_row(r8 * UNROLL + u)
                return carry

            lax.fori_loop(0, CHUNK // UNROLL, row_body, 0)

        fire_idx(0, 0)
        fire_idx(1, 1)
        wait_idx(0, 0)
        fire_gathers(0)

        def ring_body(i, carry):
            for b in range(NBUF):
                c = i * NBUF + b
                nb = (b + 1) % NBUF
                nc = c + 1

                @pl.when(nc < n_chunks)
                def _():
                    wait_idx(nc, nb)

                    @pl.when(nc >= NBUF)
                    def _():
                        wait_out(nc - NBUF, nb)

                    fire_gathers(nb)

                @pl.when(c + 2 < n_chunks)
                def _():
                    fire_idx(c + 2, (b + 2) % NBUF)

                wait_gathers(b)
                compute(rowsb[b])
                fire_out(c, b)
            return carry

        lax.fori_loop(0, n_chunks // NBUF, ring_body, 0)
        for b in range(NBUF):
            wait_out(n_chunks - NBUF + b, b)

    return sc_kernel


def kernel(x, table, gamma, beta):
    b, l = x.shape
    n_rows = b * l
    info = plsc.get_sparse_core_info()
    n_workers = info.num_cores * info.num_subcores
    x2d = x.reshape(n_rows // IDX_SUB, IDX_SUB).astype(jnp.int32)
    sc = _make_sc_kernel(n_rows, n_workers)
    out = sc(x2d, table, gamma, beta)
    return out.reshape(b, l, EMBED_D)
